# baseline (speedup 1.0000x reference)
import functools

import jax
import jax.numpy as jnp
from jax import lax
from jax.experimental import pallas as pl
from jax.experimental.pallas import tpu as pltpu
from jax.experimental.pallas import tpu_sc as plsc

_N = 10000
_D = 128
_CHUNK = 128
_NC = 2
_NS = 16
_NW = _NC * _NS
_ACC_ROWS = 10240
_RPT = _ACC_ROWS // _NS
_NCH_FULL = _N // _CHUNK
_TAIL = _N - _NCH_FULL * _CHUNK


_BLK = 8
_R0 = 1.0


def _build_msum(cpt0, cpt1):
  assert cpt0 % _BLK == 0 and cpt1 % _BLK == 0
  mesh = plsc.VectorSubcoreMesh(core_axis_name="c", subcore_axis_name="s")
  scratch = (
      pltpu.VMEM((_BLK, _CHUNK), jnp.int32),
      pltpu.VMEM((_BLK, _CHUNK), jnp.int32),
      pltpu.VMEM((_CHUNK, _D), jnp.float32),
      pltpu.VMEM((_CHUNK, _D), jnp.float32),
      pltpu.VMEM_SHARED((_ACC_ROWS, _D), jnp.float32),
      pltpu.SemaphoreType.DMA,
      pltpu.SemaphoreType.DMA,
      pltpu.SemaphoreType.DMA,
      pltpu.SemaphoreType.DMA,
  )

  @functools.partial(
      pl.kernel, mesh=mesh, scratch_types=scratch,
      out_type=(jax.ShapeDtypeStruct((_NC, _N, _D), jnp.float32),))
  def k(h_hbm, src_hbm, dst_hbm, zrow_hbm, msum_hbm,
        sblk, dblk, rows0, rows1, acc, sg0, sg1, ss0, ss1):
    c = lax.axis_index("c")
    s = lax.axis_index("s")
    rows = (rows0, rows1)
    sg = (sg0, sg1)
    ss = (ss0, ss1)
    nblk = jnp.where(c == 0, cpt0 // _BLK, cpt1 // _BLK)
    tilebase = jnp.where(c == 0, s * cpt0, _NS * cpt0 + s * cpt1)

    pltpu.sync_copy(zrow_hbm, rows0)

    def zero_body(j, carry):
      r = s * _RPT + j * _CHUNK
      pltpu.sync_copy(rows0, acc.at[pl.ds(r, _CHUNK)])
      return carry
    lax.fori_loop(0, _RPT // _CHUNK, zero_body, 0)

    plsc.subcore_barrier()

    def blk_body(B, carry):
      rowbase = tilebase + B * _BLK
      pltpu.sync_copy(src_hbm.at[pl.ds(rowbase, _BLK)], sblk)
      pltpu.sync_copy(dst_hbm.at[pl.ds(rowbase, _BLK)], dblk)
      gd = [None, None]
      sd = [None, None]
      gd[0] = pltpu.async_copy(h_hbm.at[sblk.at[0]], rows0, sg0)
      for b in range(_BLK):
        buf = b % 2
        other = 1 - buf
        if b + 1 < _BLK:
          if b >= 1:
            sd[other].wait()
          gd[other] = pltpu.async_copy(
              h_hbm.at[sblk.at[b + 1]], rows[other], sg[other])
        gd[buf].wait()
        sd[buf] = pltpu.async_copy(
            rows[buf], acc.at[dblk.at[b]], ss[buf], add=True)
      sd[0].wait()
      sd[1].wait()
      return carry
    lax.fori_loop(0, nblk, blk_body, 0)

    plsc.subcore_barrier()

    for k_ in range((_NCH_FULL + _NS) // _NS):
      idx = s + k_ * _NS

      @pl.when(idx < _NCH_FULL)
      def _full_chunk():
        r = idx * _CHUNK
        pltpu.sync_copy(acc.at[pl.ds(r, _CHUNK)], rows0)
        pltpu.sync_copy(rows0, msum_hbm.at[c, pl.ds(r, _CHUNK)])

      @pl.when(idx == _NCH_FULL)
      def _tail_chunk():
        r = _NCH_FULL * _CHUNK
        pltpu.sync_copy(acc.at[pl.ds(r, _TAIL)], rows0.at[pl.ds(0, _TAIL)])
        pltpu.sync_copy(rows0.at[pl.ds(0, _TAIL)],
                        msum_hbm.at[c, pl.ds(r, _TAIL)])

  return k


def _build_count(cpt):
  assert cpt % _BLK == 0
  nblk = cpt // _BLK
  mesh = plsc.VectorSubcoreMesh(core_axis_name="c", subcore_axis_name="s")
  scratch = (
      pltpu.VMEM((_BLK, _CHUNK), jnp.int32),
      pltpu.VMEM((_CHUNK, _D), jnp.float32),
      pltpu.VMEM((_CHUNK, _D), jnp.float32),
      pltpu.VMEM_SHARED((_ACC_ROWS, _D), jnp.float32),
      pltpu.SemaphoreType.DMA,
  )

  @functools.partial(
      pl.kernel, mesh=mesh, scratch_types=scratch,
      out_type=(jax.ShapeDtypeStruct((_NC, _N, _D), jnp.float32),))
  def k(dst_hbm, ones_hbm, zcnt_hbm, cnt_hbm, dblk, ones, cbuf, cacc, sem):
    c = lax.axis_index("c")
    s = lax.axis_index("s")
    wid = s * _NC + c

    pltpu.sync_copy(ones_hbm, ones)
    pltpu.sync_copy(zcnt_hbm, cbuf)

    def zero_body(j, carry):
      r = s * _RPT + j * _CHUNK
      pltpu.sync_copy(cbuf, cacc.at[pl.ds(r, _CHUNK)])
      return carry
    lax.fori_loop(0, _RPT // _CHUNK, zero_body, 0)

    plsc.subcore_barrier()

    def blk_body(B, carry):
      rowbase = wid * cpt + B * _BLK
      pltpu.sync_copy(dst_hbm.at[pl.ds(rowbase, _BLK)], dblk)
      descs = [pltpu.async_copy(ones, cacc.at[dblk.at[b]], sem, add=True)
               for b in range(_BLK)]
      for d_ in descs:
        d_.wait()
      return carry
    lax.fori_loop(0, nblk, blk_body, 0)

    plsc.subcore_barrier()

    for k_ in range((_NCH_FULL + _NS) // _NS):
      idx = s + k_ * _NS

      @pl.when(idx < _NCH_FULL)
      def _full_chunk():
        r = idx * _CHUNK
        pltpu.sync_copy(cacc.at[pl.ds(r, _CHUNK)], cbuf)
        pltpu.sync_copy(cbuf, cnt_hbm.at[c, pl.ds(r, _CHUNK)])

      @pl.when(idx == _NCH_FULL)
      def _tail_chunk():
        r = _NCH_FULL * _CHUNK
        pltpu.sync_copy(cacc.at[pl.ds(r, _TAIL)], cbuf.at[pl.ds(0, _TAIL)])
        pltpu.sync_copy(cbuf.at[pl.ds(0, _TAIL)],
                        cnt_hbm.at[c, pl.ds(r, _TAIL)])

  return k


def _combine(p, cnt, h, Wl, Wr, b, g, be):
  def body(p_ref, cnt_ref, h_ref, wl_ref, wr_ref, b_ref, g_ref, be_ref, o_ref):
    msum = p_ref[0, :, :] + p_ref[1, :, :]
    n_edges = cnt_ref[0, :, 0:1] + cnt_ref[1, :, 0:1]
    mean = msum / jnp.maximum(n_edges, 1.0)
    t = (jnp.dot(mean, wl_ref[...], preferred_element_type=jnp.float32)
         + jnp.dot(h_ref[...], wr_ref[...], preferred_element_type=jnp.float32)
         + b_ref[...])
    mu = jnp.mean(t, axis=0, keepdims=True)
    var = jnp.mean(jnp.square(t - mu), axis=0, keepdims=True)
    o_ref[...] = jnp.maximum(
        (t - mu) * lax.rsqrt(var + 1e-5) * g_ref[...] + be_ref[...], 0.0)

  return pl.pallas_call(
      body, out_shape=jax.ShapeDtypeStruct((_N, _D), jnp.float32),
  )(p, cnt, h, Wl, Wr, b.reshape(1, _D), g.reshape(1, _D), be.reshape(1, _D))


def _head(h1, h2, h3, w1a, w1b, w1c, bc1p, w2p, bc2p):
  def body(h1_ref, h2_ref, h3_ref, a_ref, b_ref, c_ref, bc1_ref, w2_ref,
           bc2_ref, o_ref):
    z = (jnp.dot(h1_ref[...], a_ref[...], preferred_element_type=jnp.float32)
         + jnp.dot(h2_ref[...], b_ref[...], preferred_element_type=jnp.float32)
         + jnp.dot(h3_ref[...], c_ref[...], preferred_element_type=jnp.float32)
         + bc1_ref[...])
    z = jnp.maximum(z, 0.0)
    o_ref[...] = (jnp.dot(z, w2_ref[...], preferred_element_type=jnp.float32)
                  + bc2_ref[...])

  return pl.pallas_call(
      body, out_shape=jax.ShapeDtypeStruct((_N, 128), jnp.float32),
  )(h1, h2, h3, w1a, w1b, w1c, bc1p, w2p, bc2p)


def kernel(x, edge_index, Wl0, Wr0, b0, g0, be0, Wl1, Wr1, b1, g1, be1,
           Wl2, Wr2, b2, g2, be2, Wc1, bc1, Wc2, bc2):
  E = edge_index.shape[1]
  cpt = -(-E // (_NW * _CHUNK))
  cpt = -(-cpt // _BLK) * _BLK
  E2 = cpt * _NW * _CHUNK
  src = edge_index[0]
  dst = edge_index[1]
  if E2 != E:
    pad = E2 - E
    src = jnp.concatenate([src, jnp.zeros((pad,), jnp.int32)])
    dst = jnp.concatenate([dst, jnp.full((pad,), _N, jnp.int32)])
  src = src.reshape(E2 // _CHUNK, _CHUNK)
  dst = dst.reshape(E2 // _CHUNK, _CHUNK)

  zrow = jnp.zeros((_CHUNK, _D), jnp.float32)
  ones128 = jnp.ones((_CHUNK, _D), jnp.float32)

  cpt0 = max(_BLK, int(round(2 * cpt * _R0 / _BLK)) * _BLK)
  cpt1 = 2 * cpt - cpt0
  msum_k = _build_msum(cpt0, cpt1)
  count_k = _build_count(cpt)

  (cnt,) = count_k(dst, ones128, zrow)
  (p0,) = msum_k(x, src, dst, zrow)
  h1 = _combine(p0, cnt, x, Wl0, Wr0, b0, g0, be0)
  (p1,) = msum_k(h1, src, dst, zrow)
  h2 = _combine(p1, cnt, h1, Wl1, Wr1, b1, g1, be1)
  (p2,) = msum_k(h2, src, dst, zrow)
  h3 = _combine(p2, cnt, h2, Wl2, Wr2, b2, g2, be2)

  hh = Wc1.shape[1]
  w1a = jnp.pad(Wc1[0:_D], ((0, 0), (0, 128 - hh)))
  w1b = jnp.pad(Wc1[_D:2 * _D], ((0, 0), (0, 128 - hh)))
  w1c = jnp.pad(Wc1[2 * _D:3 * _D], ((0, 0), (0, 128 - hh)))
  bc1p = jnp.pad(bc1.reshape(1, hh), ((0, 0), (0, 128 - hh)))
  w2p = jnp.pad(Wc2, ((0, 128 - hh), (0, 128 - Wc2.shape[1])))
  bc2p = jnp.pad(bc2.reshape(1, -1), ((0, 0), (0, 128 - Wc2.shape[1])))

  out = _head(h1, h2, h3, w1a, w1b, w1c, bc1p, w2p, bc2p)
  return out[:, :Wc2.shape[1]]


# speedup vs baseline: 3.3301x; 3.3301x over previous
import functools

import jax
import jax.numpy as jnp
from jax import lax
from jax.experimental import pallas as pl
from jax.experimental.pallas import tpu as pltpu
from jax.experimental.pallas import tpu_sc as plsc

_N = 10000
_D = 128
_CHUNK = 128
_NC = 2
_NS = 16
_NW = _NC * _NS
_ACC_ROWS = 10240
_RPT = _ACC_ROWS // _NS
_NCH_FULL = _N // _CHUNK
_TAIL = _N - _NCH_FULL * _CHUNK


_BLK = 8
_R0 = 0.5


def _build_msum(cpt0, cpt1):
  assert cpt0 % _BLK == 0 and cpt1 % _BLK == 0
  mesh = plsc.VectorSubcoreMesh(core_axis_name="c", subcore_axis_name="s")
  scratch = (
      pltpu.VMEM((_BLK, _CHUNK), jnp.int32),
      pltpu.VMEM((_BLK, _CHUNK), jnp.int32),
      pltpu.VMEM((_CHUNK, _D), jnp.float32),
      pltpu.VMEM((_CHUNK, _D), jnp.float32),
      pltpu.VMEM_SHARED((_ACC_ROWS, _D), jnp.float32),
      pltpu.SemaphoreType.DMA,
      pltpu.SemaphoreType.DMA,
      pltpu.SemaphoreType.DMA,
      pltpu.SemaphoreType.DMA,
  )

  @functools.partial(
      pl.kernel, mesh=mesh, scratch_types=scratch,
      out_type=(jax.ShapeDtypeStruct((_NC, _N, _D), jnp.float32),))
  def k(h_hbm, src_hbm, dst_hbm, zrow_hbm, msum_hbm,
        sblk, dblk, rows0, rows1, acc, sg0, sg1, ss0, ss1):
    c = lax.axis_index("c")
    s = lax.axis_index("s")
    rows = (rows0, rows1)
    sg = (sg0, sg1)
    ss = (ss0, ss1)
    nblk = jnp.where(c == 0, cpt0 // _BLK, cpt1 // _BLK)
    tilebase = jnp.where(c == 0, s * cpt0, _NS * cpt0 + s * cpt1)

    pltpu.sync_copy(zrow_hbm, rows0)

    def zero_body(j, carry):
      r = s * _RPT + j * _CHUNK
      pltpu.sync_copy(rows0, acc.at[pl.ds(r, _CHUNK)])
      return carry
    lax.fori_loop(0, _RPT // _CHUNK, zero_body, 0)

    plsc.subcore_barrier()

    def blk_body(B, carry):
      rowbase = tilebase + B * _BLK
      pltpu.sync_copy(src_hbm.at[pl.ds(rowbase, _BLK)], sblk)
      pltpu.sync_copy(dst_hbm.at[pl.ds(rowbase, _BLK)], dblk)
      gd = [None, None]
      sd = [None, None]
      gd[0] = pltpu.async_copy(h_hbm.at[sblk.at[0]], rows0, sg0)
      for b in range(_BLK):
        buf = b % 2
        other = 1 - buf
        if b + 1 < _BLK:
          if b >= 1:
            sd[other].wait()
          gd[other] = pltpu.async_copy(
              h_hbm.at[sblk.at[b + 1]], rows[other], sg[other])
        gd[buf].wait()
        sd[buf] = pltpu.async_copy(
            rows[buf], acc.at[dblk.at[b]], ss[buf], add=True)
      sd[0].wait()
      sd[1].wait()
      return carry
    lax.fori_loop(0, nblk, blk_body, 0)

    plsc.subcore_barrier()

    for k_ in range((_NCH_FULL + _NS) // _NS):
      idx = s + k_ * _NS

      @pl.when(idx < _NCH_FULL)
      def _full_chunk():
        r = idx * _CHUNK
        pltpu.sync_copy(acc.at[pl.ds(r, _CHUNK)], rows0)
        pltpu.sync_copy(rows0, msum_hbm.at[c, pl.ds(r, _CHUNK)])

      @pl.when(idx == _NCH_FULL)
      def _tail_chunk():
        r = _NCH_FULL * _CHUNK
        pltpu.sync_copy(acc.at[pl.ds(r, _TAIL)], rows0.at[pl.ds(0, _TAIL)])
        pltpu.sync_copy(rows0.at[pl.ds(0, _TAIL)],
                        msum_hbm.at[c, pl.ds(r, _TAIL)])

  return k


def _build_count(cpt):
  assert cpt % _BLK == 0
  nblk = cpt // _BLK
  mesh = plsc.VectorSubcoreMesh(core_axis_name="c", subcore_axis_name="s")
  scratch = (
      pltpu.VMEM((_BLK, _CHUNK), jnp.int32),
      pltpu.VMEM((_CHUNK, _D), jnp.float32),
      pltpu.VMEM((_CHUNK, _D), jnp.float32),
      pltpu.VMEM_SHARED((_ACC_ROWS, _D), jnp.float32),
      pltpu.SemaphoreType.DMA,
  )

  @functools.partial(
      pl.kernel, mesh=mesh, scratch_types=scratch,
      out_type=(jax.ShapeDtypeStruct((_NC, _N, _D), jnp.float32),))
  def k(dst_hbm, ones_hbm, zcnt_hbm, cnt_hbm, dblk, ones, cbuf, cacc, sem):
    c = lax.axis_index("c")
    s = lax.axis_index("s")
    wid = s * _NC + c

    pltpu.sync_copy(ones_hbm, ones)
    pltpu.sync_copy(zcnt_hbm, cbuf)

    def zero_body(j, carry):
      r = s * _RPT + j * _CHUNK
      pltpu.sync_copy(cbuf, cacc.at[pl.ds(r, _CHUNK)])
      return carry
    lax.fori_loop(0, _RPT // _CHUNK, zero_body, 0)

    plsc.subcore_barrier()

    def blk_body(B, carry):
      rowbase = wid * cpt + B * _BLK
      pltpu.sync_copy(dst_hbm.at[pl.ds(rowbase, _BLK)], dblk)
      descs = [pltpu.async_copy(ones, cacc.at[dblk.at[b]], sem, add=True)
               for b in range(_BLK)]
      for d_ in descs:
        d_.wait()
      return carry
    lax.fori_loop(0, nblk, blk_body, 0)

    plsc.subcore_barrier()

    for k_ in range((_NCH_FULL + _NS) // _NS):
      idx = s + k_ * _NS

      @pl.when(idx < _NCH_FULL)
      def _full_chunk():
        r = idx * _CHUNK
        pltpu.sync_copy(cacc.at[pl.ds(r, _CHUNK)], cbuf)
        pltpu.sync_copy(cbuf, cnt_hbm.at[c, pl.ds(r, _CHUNK)])

      @pl.when(idx == _NCH_FULL)
      def _tail_chunk():
        r = _NCH_FULL * _CHUNK
        pltpu.sync_copy(cacc.at[pl.ds(r, _TAIL)], cbuf.at[pl.ds(0, _TAIL)])
        pltpu.sync_copy(cbuf.at[pl.ds(0, _TAIL)],
                        cnt_hbm.at[c, pl.ds(r, _TAIL)])

  return k


def _combine(p, cnt, h, Wl, Wr, b, g, be):
  def body(p_ref, cnt_ref, h_ref, wl_ref, wr_ref, b_ref, g_ref, be_ref, o_ref):
    msum = p_ref[0, :, :] + p_ref[1, :, :]
    n_edges = cnt_ref[0, :, 0:1] + cnt_ref[1, :, 0:1]
    mean = msum / jnp.maximum(n_edges, 1.0)
    t = (jnp.dot(mean, wl_ref[...], preferred_element_type=jnp.float32)
         + jnp.dot(h_ref[...], wr_ref[...], preferred_element_type=jnp.float32)
         + b_ref[...])
    mu = jnp.mean(t, axis=0, keepdims=True)
    var = jnp.mean(jnp.square(t - mu), axis=0, keepdims=True)
    o_ref[...] = jnp.maximum(
        (t - mu) * lax.rsqrt(var + 1e-5) * g_ref[...] + be_ref[...], 0.0)

  return pl.pallas_call(
      body, out_shape=jax.ShapeDtypeStruct((_N, _D), jnp.float32),
  )(p, cnt, h, Wl, Wr, b.reshape(1, _D), g.reshape(1, _D), be.reshape(1, _D))


def _head(h1, h2, h3, w1a, w1b, w1c, bc1p, w2p, bc2p):
  def body(h1_ref, h2_ref, h3_ref, a_ref, b_ref, c_ref, bc1_ref, w2_ref,
           bc2_ref, o_ref):
    z = (jnp.dot(h1_ref[...], a_ref[...], preferred_element_type=jnp.float32)
         + jnp.dot(h2_ref[...], b_ref[...], preferred_element_type=jnp.float32)
         + jnp.dot(h3_ref[...], c_ref[...], preferred_element_type=jnp.float32)
         + bc1_ref[...])
    z = jnp.maximum(z, 0.0)
    o_ref[...] = (jnp.dot(z, w2_ref[...], preferred_element_type=jnp.float32)
                  + bc2_ref[...])

  return pl.pallas_call(
      body, out_shape=jax.ShapeDtypeStruct((_N, 128), jnp.float32),
  )(h1, h2, h3, w1a, w1b, w1c, bc1p, w2p, bc2p)


def kernel(x, edge_index, Wl0, Wr0, b0, g0, be0, Wl1, Wr1, b1, g1, be1,
           Wl2, Wr2, b2, g2, be2, Wc1, bc1, Wc2, bc2):
  E = edge_index.shape[1]
  cpt = -(-E // (_NW * _CHUNK))
  cpt = -(-cpt // _BLK) * _BLK
  E2 = cpt * _NW * _CHUNK
  src = edge_index[0]
  dst = edge_index[1]
  if E2 != E:
    pad = E2 - E
    i = jnp.arange(pad, dtype=jnp.int32)
    src = jnp.concatenate([src, i % _N])
    dst = jnp.concatenate([dst, _N + i % (_ACC_ROWS - _N)])
  src = src.reshape(E2 // _CHUNK, _CHUNK)
  dst = dst.reshape(E2 // _CHUNK, _CHUNK)

  zrow = jnp.zeros((_CHUNK, _D), jnp.float32)
  ones128 = jnp.ones((_CHUNK, _D), jnp.float32)

  cpt0 = max(_BLK, int(round(2 * cpt * _R0 / _BLK)) * _BLK)
  cpt1 = 2 * cpt - cpt0
  msum_k = _build_msum(cpt0, cpt1)
  count_k = _build_count(cpt)

  (cnt,) = count_k(dst, ones128, zrow)
  (p0,) = msum_k(x, src, dst, zrow)
  h1 = _combine(p0, cnt, x, Wl0, Wr0, b0, g0, be0)
  (p1,) = msum_k(h1, src, dst, zrow)
  h2 = _combine(p1, cnt, h1, Wl1, Wr1, b1, g1, be1)
  (p2,) = msum_k(h2, src, dst, zrow)
  h3 = _combine(p2, cnt, h2, Wl2, Wr2, b2, g2, be2)

  hh = Wc1.shape[1]
  w1a = jnp.pad(Wc1[0:_D], ((0, 0), (0, 128 - hh)))
  w1b = jnp.pad(Wc1[_D:2 * _D], ((0, 0), (0, 128 - hh)))
  w1c = jnp.pad(Wc1[2 * _D:3 * _D], ((0, 0), (0, 128 - hh)))
  bc1p = jnp.pad(bc1.reshape(1, hh), ((0, 0), (0, 128 - hh)))
  w2p = jnp.pad(Wc2, ((0, 128 - hh), (0, 128 - Wc2.shape[1])))
  bc2p = jnp.pad(bc2.reshape(1, -1), ((0, 0), (0, 128 - Wc2.shape[1])))

  out = _head(h1, h2, h3, w1a, w1b, w1c, bc1p, w2p, bc2p)
  return out[:, :Wc2.shape[1]]
